# baseline (speedup 1.0000x reference)
import dataclasses
import functools

import jax
import jax.numpy as jnp
from jax import lax
from jax.experimental import pallas as pl
from jax.experimental.pallas import tpu as pltpu
from jax.experimental.pallas import tpu_sc as plsc

N = 10000
INC = 160000
H = 256
HH = 128
NSUB = 16
PER_TILE = INC // NSUB
CH = 80
NFULL = PER_TILE // CH
RQ = 4
SQ = 8
ROWS0 = 632
ROWSL = N - (NSUB - 1) * ROWS0
F32 = jnp.float32


def _mesh():
    return plsc.VectorSubcoreMesh(core_axis_name="c", subcore_axis_name="s",
                                  num_cores=2, num_subcores=NSUB)


def _seg_pass(src, gidx, sidx):
    @functools.partial(
        pl.kernel,
        out_type=jax.ShapeDtypeStruct((2, N, HH), F32),
        mesh=_mesh(),
        scratch_types=[
            pltpu.VMEM((RQ, CH, HH), F32),
            pltpu.VMEM((SQ, CH), jnp.int32),
            pltpu.VMEM((SQ, CH), jnp.int32),
            pltpu.VMEM_SHARED((N, HH), F32),
            pltpu.SemaphoreType.DMA((RQ,)),
            pltpu.SemaphoreType.DMA((RQ,)),
            pltpu.SemaphoreType.DMA((SQ,)),
        ],
    )
    def k(gidx_hbm, sidx_hbm, src_hbm, dst_hbm, rows, gbuf, sbuf,
          acc, sem_g, sem_s, sem_i):
        c = lax.axis_index("c")
        s = lax.axis_index("s")
        z16 = jnp.zeros((16,), F32)

        def phase_all(cc):
            base = s * PER_TILE

            def idx_cp(j, m):
                return (pltpu.make_async_copy(
                            gidx_hbm.at[pl.ds(base + j * CH, CH)],
                            gbuf.at[m], sem_i.at[m]),
                        pltpu.make_async_copy(
                            sidx_hbm.at[pl.ds(base + j * CH, CH)],
                            sbuf.at[m], sem_i.at[m]))

            def gat_cp(m8, m4):
                return pltpu.make_async_copy(src_hbm.at[cc].at[gbuf.at[m8]],
                                             rows.at[m4], sem_g.at[m4])

            def idx_start(j, m):
                a, b = idx_cp(j, m)
                a.start()
                b.start()

            def idx_wait(j, m):
                a, b = idx_cp(j, m)
                a.wait()
                b.wait()

            def sct_start(m8, m4):
                pltpu.async_copy(rows.at[m4], acc.at[sbuf.at[m8]],
                                 sem_s.at[m4], add=True)

            def sct_wait(m8, m4):
                pltpu.make_async_copy(rows.at[m4], acc.at[sbuf.at[m8]],
                                      sem_s.at[m4]).wait()

            def sched(kk, m8, head=False):
                m4 = m8 % 4
                gat_cp(m8, m4).wait()
                if not (head and isinstance(kk, int) and kk < 2):
                    sct_wait((m8 - 2) % SQ, (m4 - 2) % RQ)
                if not (isinstance(kk, int) and kk + 2 >= NFULL):
                    idx_wait(kk + 2, (m8 + 2) % SQ)
                    gat_cp((m8 + 2) % SQ, (m4 + 2) % RQ).start()
                if not (isinstance(kk, int) and kk + 4 >= NFULL):
                    idx_start(kk + 4, (m8 + 4) % SQ)
                sct_start(m8, m4)

            for j in range(4):
                idx_start(j, j)
            for j in range(2):
                idx_wait(j, j)
                gat_cp(j, j).start()

            @pl.loop(0, CH)
            def _(r):
                @pl.loop(0, HH, step=16)
                def _(j):
                    rows[RQ - 1, r, pl.ds(j, 16)] = z16

            def zero_stripe(roff, rlen):
                nf, rem = rlen // CH, rlen % CH
                for t in range(nf):
                    pltpu.sync_copy(rows.at[RQ - 1],
                                    acc.at[pl.ds(roff + t * CH, CH)])
                if rem:
                    pltpu.sync_copy(rows.at[RQ - 1].at[pl.ds(0, rem)],
                                    acc.at[pl.ds(roff + nf * CH, rem)])

            @pl.when(s < NSUB - 1)
            def _():
                zero_stripe(s * ROWS0, ROWS0)

            @pl.when(s == NSUB - 1)
            def _():
                zero_stripe((NSUB - 1) * ROWS0, ROWSL)

            plsc.subcore_barrier()

            for kk in range(8):
                sched(kk, kk, head=True)

            @pl.loop(8, 120, step=8)
            def _(k0):
                for d in range(8):
                    sched(k0 + d, d)

            for kk in range(120, NFULL):
                sched(kk, kk % SQ)

            sct_wait((NFULL - 2) % SQ, (NFULL - 2) % RQ)
            sct_wait((NFULL - 1) % SQ, (NFULL - 1) % RQ)

            plsc.subcore_barrier()

            @pl.when(s < NSUB - 1)
            def _():
                pltpu.sync_copy(acc.at[pl.ds(s * ROWS0, ROWS0)],
                                dst_hbm.at[cc].at[pl.ds(s * ROWS0, ROWS0)])

            @pl.when(s == NSUB - 1)
            def _():
                pltpu.sync_copy(
                    acc.at[pl.ds((NSUB - 1) * ROWS0, ROWSL)],
                    dst_hbm.at[cc].at[pl.ds((NSUB - 1) * ROWS0, ROWSL)])

        @pl.when(c == 0)
        def _():
            phase_all(0)

        @pl.when(c == 1)
        def _():
            phase_all(1)

    return k(gidx, sidx, src)


def _degrees(nidx, eidx, w):
    NPAD = 10240
    SPAN = NPAD // NSUB
    LASTD = N - (NSUB - 1) * SPAN

    cp = pltpu.CompilerParams()
    if "needs_layout_passes" in pltpu.CompilerParams.__dataclass_fields__:
        cp = dataclasses.replace(cp, needs_layout_passes=False)

    @functools.partial(
        pl.kernel,
        out_type=(jax.ShapeDtypeStruct((N, HH), F32),
                  jax.ShapeDtypeStruct((N, HH), F32)),
        mesh=_mesh(),
        compiler_params=cp,
        scratch_types=[
            pltpu.VMEM((PER_TILE,), jnp.int32),
            pltpu.VMEM((PER_TILE,), jnp.int32),
            pltpu.VMEM((N,), F32),
            pltpu.VMEM((NPAD,), F32),
            pltpu.VMEM_SHARED((NSUB, NPAD), F32),
            pltpu.VMEM((SPAN,), F32),
            pltpu.VMEM((SPAN,), F32),
            pltpu.VMEM((128, HH), F32),
        ],
    )
    def k(nidx_hbm, eidx_hbm, w_hbm, dinv_hbm, binv_hbm,
          ebuf, nbuf, wbuf, acc, stage, rbuf, abuf, rep):
        c = lax.axis_index("c")
        s = lax.axis_index("s")
        z16 = jnp.zeros((16,), F32)
        base = s * PER_TILE
        pltpu.sync_copy(eidx_hbm.at[pl.ds(base, PER_TILE)], ebuf)

        @pl.loop(0, NPAD, step=16)
        def _(i):
            acc[pl.ds(i, 16)] = z16

        @pl.when(c == 0)
        def _():
            pltpu.sync_copy(nidx_hbm.at[pl.ds(base, PER_TILE)], nbuf)
            pltpu.sync_copy(w_hbm, wbuf)

            @pl.loop(0, PER_TILE, step=16)
            def _(i):
                e = ebuf[pl.ds(i, 16)]
                n = nbuf[pl.ds(i, 16)]
                wv = plsc.load_gather(wbuf, [e])
                plsc.addupdate_scatter(acc, [n], wv)

        @pl.when(c == 1)
        def _():
            o16 = jnp.full((16,), 1.0, F32)

            @pl.loop(0, PER_TILE, step=16)
            def _(i):
                e = ebuf[pl.ds(i, 16)]
                plsc.addupdate_scatter(acc, [e], o16)

        pltpu.sync_copy(acc, stage.at[s])
        plsc.subcore_barrier()

        def reduce_span(off, drain_len, out_hbm):
            @pl.loop(0, SPAN, step=16)
            def _(j):
                abuf[pl.ds(j, 16)] = z16

            @pl.loop(0, NSUB)
            def _(r):
                pltpu.sync_copy(stage.at[r].at[pl.ds(off, SPAN)], rbuf)

                @pl.loop(0, SPAN, step=16)
                def _(j):
                    abuf[pl.ds(j, 16)] = abuf[pl.ds(j, 16)] + rbuf[pl.ds(j, 16)]

            @pl.loop(0, SPAN, step=16)
            def _(j):
                v = abuf[pl.ds(j, 16)]
                zm = v == 0.0
                abuf[pl.ds(j, 16)] = jnp.where(zm, 0.0,
                                               1.0 / jnp.where(zm, 1.0, v))

            nfc, remc = drain_len // 128, drain_len % 128
            for t in range(nfc + (1 if remc else 0)):
                rl = 128 if t < nfc else remc

                @pl.loop(0, rl // 16)
                def _(q):
                    av = abuf[pl.ds(t * 128 + q * 16, 16)]
                    for ln in range(16):
                        vv = jnp.full((16,), 1.0, F32) * av[ln]

                        @pl.loop(0, HH, step=16)
                        def _(j):
                            rep[q * 16 + ln, pl.ds(j, 16)] = vv

                pltpu.sync_copy(rep.at[pl.ds(0, rl)],
                                out_hbm.at[pl.ds(off + t * 128, rl)])

        for cc, out_hbm in ((0, dinv_hbm), (1, binv_hbm)):
            @pl.when(jnp.logical_and(c == cc, s < NSUB - 1))
            def _():
                reduce_span(s * SPAN, SPAN, out_hbm)

            @pl.when(jnp.logical_and(c == cc, s == NSUB - 1))
            def _():
                reduce_span((NSUB - 1) * SPAN, LASTD, out_hbm)

    return k(nidx, eidx, w)


_RT = 2000
_NI = N // _RT


def _dot_t(x, wb):
    return lax.dot_general(x, wb, (((1,), (1,)), ((), ())),
                           precision=lax.Precision.DEFAULT,
                           preferred_element_type=F32)


def _mm_first(emb, W):
    def body(x_ref, w_ref, out_ref):
        x = x_ref[...]
        w = w_ref[...]
        for g in range(2):
            out_ref[g] = _dot_t(x, w[g * HH:(g + 1) * HH, :])

    return pl.pallas_call(
        body,
        grid=(_NI,),
        in_specs=[pl.BlockSpec((_RT, H), lambda i: (i, 0)),
                  pl.BlockSpec((H, H), lambda i: (0, 0))],
        out_specs=pl.BlockSpec((2, _RT, HH), lambda i: (0, i, 0)),
        out_shape=jax.ShapeDtypeStruct((2, N, HH), F32),
    )(emb, W)


def _mm_mid(oh, dinv_r, b2d, W):
    def body(o_ref, d_ref, b_ref, w_ref, out_ref):
        dv = d_ref[...]
        bv = b_ref[...]
        w = w_ref[...]
        x0 = jnp.maximum(o_ref[0] * dv + bv[:, :HH], 0.0)
        x1 = jnp.maximum(o_ref[1] * dv + bv[:, HH:], 0.0)
        x = jnp.concatenate([x0, x1], axis=1)
        for g in range(2):
            out_ref[g] = _dot_t(x, w[g * HH:(g + 1) * HH, :])

    return pl.pallas_call(
        body,
        grid=(_NI,),
        in_specs=[pl.BlockSpec((2, _RT, HH), lambda i: (0, i, 0)),
                  pl.BlockSpec((_RT, HH), lambda i: (i, 0)),
                  pl.BlockSpec((1, H), lambda i: (0, 0)),
                  pl.BlockSpec((H, H), lambda i: (0, 0))],
        out_specs=pl.BlockSpec((2, _RT, HH), lambda i: (0, i, 0)),
        out_shape=jax.ShapeDtypeStruct((2, N, HH), F32),
    )(oh, dinv_r, b2d, W)


def _scale(e, binv_r):
    def body(e_ref, s_ref, out_ref):
        out_ref[...] = e_ref[...] * s_ref[...][None, :, :]

    return pl.pallas_call(
        body,
        grid=(_NI,),
        in_specs=[pl.BlockSpec((2, _RT, HH), lambda i: (0, i, 0)),
                  pl.BlockSpec((_RT, HH), lambda i: (i, 0))],
        out_specs=pl.BlockSpec((2, _RT, HH), lambda i: (0, i, 0)),
        out_shape=jax.ShapeDtypeStruct((2, N, HH), F32),
    )(e, binv_r)


def _final(oh, dinv_r, b2d):
    def body(o_ref, d_ref, b_ref, out_ref):
        dv = d_ref[...]
        bv = b_ref[...]
        out_ref[:, :HH] = o_ref[0] * dv + bv[:, :HH]
        out_ref[:, HH:] = o_ref[1] * dv + bv[:, HH:]

    return pl.pallas_call(
        body,
        grid=(_NI,),
        in_specs=[pl.BlockSpec((2, _RT, HH), lambda i: (0, i, 0)),
                  pl.BlockSpec((_RT, HH), lambda i: (i, 0)),
                  pl.BlockSpec((1, H), lambda i: (0, 0))],
        out_specs=pl.BlockSpec((_RT, H), lambda i: (i, 0)),
        out_shape=jax.ShapeDtypeStruct((N, H), F32),
    )(oh, dinv_r, b2d)


def kernel(hyperedge_index, hyperedge_weight, embedding, W0, b0, W1, b1, W2,
           b2):
    nidx = hyperedge_index[0]
    eidx = hyperedge_index[1]
    dinv_r, binv_r = _degrees(nidx, eidx, hyperedge_weight)

    y = _mm_first(embedding, W0)
    for Wn, bn in ((W1, b0), (W2, b1)):
        e = _seg_pass(y, nidx, eidx)
        z = _scale(e, binv_r)
        o = _seg_pass(z, eidx, nidx)
        y = _mm_mid(o, dinv_r, bn[None, :], Wn)
    e = _seg_pass(y, nidx, eidx)
    z = _scale(e, binv_r)
    o = _seg_pass(z, eidx, nidx)
    return _final(o, dinv_r, b2[None, :])


# speedup vs baseline: 1.0184x; 1.0184x over previous
import dataclasses
import functools

import jax
import jax.numpy as jnp
from jax import lax
from jax.experimental import pallas as pl
from jax.experimental.pallas import tpu as pltpu
from jax.experimental.pallas import tpu_sc as plsc

N = 10000
INC = 160000
H = 256
HH = 128
NSUB = 16
PER_TILE = INC // NSUB
CH = 80
NFULL = PER_TILE // CH
RQ = 4
SQ = 8
ROWS0 = 632
ROWSL = N - (NSUB - 1) * ROWS0
F32 = jnp.float32


def _mesh():
    return plsc.VectorSubcoreMesh(core_axis_name="c", subcore_axis_name="s",
                                  num_cores=2, num_subcores=NSUB)


def _seg_pass(src, gidx, sidx):
    @functools.partial(
        pl.kernel,
        out_type=jax.ShapeDtypeStruct((2, N, HH), F32),
        mesh=_mesh(),
        scratch_types=[
            pltpu.VMEM((RQ, CH, HH), F32),
            pltpu.VMEM((SQ, CH), jnp.int32),
            pltpu.VMEM((SQ, CH), jnp.int32),
            pltpu.VMEM_SHARED((N, HH), F32),
            pltpu.SemaphoreType.DMA((RQ,)),
            pltpu.SemaphoreType.DMA((RQ,)),
            pltpu.SemaphoreType.DMA((SQ,)),
        ],
    )
    def k(gidx_hbm, sidx_hbm, src_hbm, dst_hbm, rows, gbuf, sbuf,
          acc, sem_g, sem_s, sem_i):
        c = lax.axis_index("c")
        s = lax.axis_index("s")
        z16 = jnp.zeros((16,), F32)

        def phase_all(cc):
            base = s * PER_TILE

            def idx_cp(j, m):
                return (pltpu.make_async_copy(
                            gidx_hbm.at[pl.ds(base + j * CH, CH)],
                            gbuf.at[m], sem_i.at[m]),
                        pltpu.make_async_copy(
                            sidx_hbm.at[pl.ds(base + j * CH, CH)],
                            sbuf.at[m], sem_i.at[m]))

            def gat_cp(m8, m4):
                return pltpu.make_async_copy(src_hbm.at[cc].at[gbuf.at[m8]],
                                             rows.at[m4], sem_g.at[m4])

            def idx_start(j, m):
                a, b = idx_cp(j, m)
                a.start()
                b.start()

            def idx_wait(j, m):
                a, b = idx_cp(j, m)
                a.wait()
                b.wait()

            def sct_start(m8, m4):
                pltpu.async_copy(rows.at[m4], acc.at[sbuf.at[m8]],
                                 sem_s.at[m4], add=True)

            def sct_wait(m8, m4):
                pltpu.make_async_copy(rows.at[m4], acc.at[sbuf.at[m8]],
                                      sem_s.at[m4]).wait()

            def sched(kk, m8, head=False):
                m4 = m8 % 4
                gat_cp(m8, m4).wait()
                if not (head and isinstance(kk, int) and kk < 2):
                    sct_wait((m8 - 2) % SQ, (m4 - 2) % RQ)
                if not (isinstance(kk, int) and kk + 2 >= NFULL):
                    idx_wait(kk + 2, (m8 + 2) % SQ)
                    gat_cp((m8 + 2) % SQ, (m4 + 2) % RQ).start()
                if not (isinstance(kk, int) and kk + 4 >= NFULL):
                    idx_start(kk + 4, (m8 + 4) % SQ)
                sct_start(m8, m4)

            for j in range(4):
                idx_start(j, j)
            for j in range(2):
                idx_wait(j, j)
                gat_cp(j, j).start()

            @pl.loop(0, CH)
            def _(r):
                @pl.loop(0, HH, step=16)
                def _(j):
                    rows[RQ - 1, r, pl.ds(j, 16)] = z16

            def zero_stripe(roff, rlen):
                nf, rem = rlen // CH, rlen % CH
                for t in range(nf):
                    pltpu.sync_copy(rows.at[RQ - 1],
                                    acc.at[pl.ds(roff + t * CH, CH)])
                if rem:
                    pltpu.sync_copy(rows.at[RQ - 1].at[pl.ds(0, rem)],
                                    acc.at[pl.ds(roff + nf * CH, rem)])

            @pl.when(s < NSUB - 1)
            def _():
                zero_stripe(s * ROWS0, ROWS0)

            @pl.when(s == NSUB - 1)
            def _():
                zero_stripe((NSUB - 1) * ROWS0, ROWSL)

            plsc.subcore_barrier()

            for kk in range(8):
                sched(kk, kk, head=True)

            @pl.loop(8, 120, step=8)
            def _(k0):
                for d in range(8):
                    sched(k0 + d, d)

            for kk in range(120, NFULL):
                sched(kk, kk % SQ)

            sct_wait((NFULL - 2) % SQ, (NFULL - 2) % RQ)
            sct_wait((NFULL - 1) % SQ, (NFULL - 1) % RQ)

            plsc.subcore_barrier()

            @pl.when(s < NSUB - 1)
            def _():
                pltpu.sync_copy(acc.at[pl.ds(s * ROWS0, ROWS0)],
                                dst_hbm.at[cc].at[pl.ds(s * ROWS0, ROWS0)])

            @pl.when(s == NSUB - 1)
            def _():
                pltpu.sync_copy(
                    acc.at[pl.ds((NSUB - 1) * ROWS0, ROWSL)],
                    dst_hbm.at[cc].at[pl.ds((NSUB - 1) * ROWS0, ROWSL)])

        @pl.when(c == 0)
        def _():
            phase_all(0)

        @pl.when(c == 1)
        def _():
            phase_all(1)

    return k(gidx, sidx, src)


def _degrees(nidx, eidx, w):
    NPAD = 10240
    SPAN = NPAD // NSUB
    LASTD = N - (NSUB - 1) * SPAN

    cp = pltpu.CompilerParams()
    if "needs_layout_passes" in pltpu.CompilerParams.__dataclass_fields__:
        cp = dataclasses.replace(cp, needs_layout_passes=False)

    @functools.partial(
        pl.kernel,
        out_type=(jax.ShapeDtypeStruct((N,), F32),
                  jax.ShapeDtypeStruct((N,), F32)),
        mesh=_mesh(),
        compiler_params=cp,
        scratch_types=[
            pltpu.VMEM((PER_TILE,), jnp.int32),
            pltpu.VMEM((PER_TILE,), jnp.int32),
            pltpu.VMEM((N,), F32),
            pltpu.VMEM((NPAD,), F32),
            pltpu.VMEM_SHARED((NSUB, NPAD), F32),
            pltpu.VMEM((SPAN,), F32),
            pltpu.VMEM((SPAN,), F32),
        ],
    )
    def k(nidx_hbm, eidx_hbm, w_hbm, dinv_hbm, binv_hbm,
          ebuf, nbuf, wbuf, acc, stage, rbuf, abuf):
        c = lax.axis_index("c")
        s = lax.axis_index("s")
        z16 = jnp.zeros((16,), F32)
        base = s * PER_TILE
        pltpu.sync_copy(eidx_hbm.at[pl.ds(base, PER_TILE)], ebuf)

        @pl.loop(0, NPAD, step=16)
        def _(i):
            acc[pl.ds(i, 16)] = z16

        @pl.when(c == 0)
        def _():
            pltpu.sync_copy(nidx_hbm.at[pl.ds(base, PER_TILE)], nbuf)
            pltpu.sync_copy(w_hbm, wbuf)

            @pl.loop(0, PER_TILE, step=16)
            def _(i):
                e = ebuf[pl.ds(i, 16)]
                n = nbuf[pl.ds(i, 16)]
                wv = plsc.load_gather(wbuf, [e])
                plsc.addupdate_scatter(acc, [n], wv)

        @pl.when(c == 1)
        def _():
            o16 = jnp.full((16,), 1.0, F32)

            @pl.loop(0, PER_TILE, step=16)
            def _(i):
                e = ebuf[pl.ds(i, 16)]
                plsc.addupdate_scatter(acc, [e], o16)

        pltpu.sync_copy(acc, stage.at[s])
        plsc.subcore_barrier()

        def reduce_span(off, drain_len, out_hbm):
            @pl.loop(0, SPAN, step=16)
            def _(j):
                abuf[pl.ds(j, 16)] = z16

            @pl.loop(0, NSUB)
            def _(r):
                pltpu.sync_copy(stage.at[r].at[pl.ds(off, SPAN)], rbuf)

                @pl.loop(0, SPAN, step=16)
                def _(j):
                    abuf[pl.ds(j, 16)] = abuf[pl.ds(j, 16)] + rbuf[pl.ds(j, 16)]

            @pl.loop(0, SPAN, step=16)
            def _(j):
                v = abuf[pl.ds(j, 16)]
                zm = v == 0.0
                abuf[pl.ds(j, 16)] = jnp.where(zm, 0.0,
                                               1.0 / jnp.where(zm, 1.0, v))

            pltpu.sync_copy(abuf.at[pl.ds(0, drain_len)],
                            out_hbm.at[pl.ds(off, drain_len)])

        for cc, out_hbm in ((0, dinv_hbm), (1, binv_hbm)):
            @pl.when(jnp.logical_and(c == cc, s < NSUB - 1))
            def _():
                reduce_span(s * SPAN, SPAN, out_hbm)

            @pl.when(jnp.logical_and(c == cc, s == NSUB - 1))
            def _():
                reduce_span((NSUB - 1) * SPAN, LASTD, out_hbm)

    return k(nidx, eidx, w)


_RT = 2000
_NI = N // _RT


def _dot_t(x, wb):
    return lax.dot_general(x, wb, (((1,), (1,)), ((), ())),
                           precision=lax.Precision.DEFAULT,
                           preferred_element_type=F32)


def _mm_first(emb, W, token):
    def body(x_ref, w_ref, t_ref, out_ref):
        del t_ref
        x = x_ref[...]
        w = w_ref[...]
        for g in range(2):
            out_ref[g] = _dot_t(x, w[g * HH:(g + 1) * HH, :])

    return pl.pallas_call(
        body,
        grid=(_NI,),
        in_specs=[pl.BlockSpec((_RT, H), lambda i: (i, 0)),
                  pl.BlockSpec((H, H), lambda i: (0, 0)),
                  pl.BlockSpec((N,), lambda i: (0,))],
        out_specs=pl.BlockSpec((2, _RT, HH), lambda i: (0, i, 0)),
        out_shape=jax.ShapeDtypeStruct((2, N, HH), F32),
    )(emb, W, token)


def _mm_mid(oh, dinv_r, b2d, W):
    def body(o_ref, d_ref, b_ref, w_ref, out_ref):
        dv = d_ref[...]
        bv = b_ref[...]
        w = w_ref[...]
        x0 = jnp.maximum(o_ref[0] * dv + bv[:, :HH], 0.0)
        x1 = jnp.maximum(o_ref[1] * dv + bv[:, HH:], 0.0)
        x = jnp.concatenate([x0, x1], axis=1)
        for g in range(2):
            out_ref[g] = _dot_t(x, w[g * HH:(g + 1) * HH, :])

    return pl.pallas_call(
        body,
        grid=(_NI,),
        in_specs=[pl.BlockSpec((2, _RT, HH), lambda i: (0, i, 0)),
                  pl.BlockSpec((_RT, 1), lambda i: (i, 0)),
                  pl.BlockSpec((1, H), lambda i: (0, 0)),
                  pl.BlockSpec((H, H), lambda i: (0, 0))],
        out_specs=pl.BlockSpec((2, _RT, HH), lambda i: (0, i, 0)),
        out_shape=jax.ShapeDtypeStruct((2, N, HH), F32),
    )(oh, dinv_r, b2d, W)


def _scale(e, binv_r):
    def body(e_ref, s_ref, out_ref):
        out_ref[...] = e_ref[...] * s_ref[...][None, :, :]

    return pl.pallas_call(
        body,
        grid=(_NI,),
        in_specs=[pl.BlockSpec((2, _RT, HH), lambda i: (0, i, 0)),
                  pl.BlockSpec((_RT, 1), lambda i: (i, 0))],
        out_specs=pl.BlockSpec((2, _RT, HH), lambda i: (0, i, 0)),
        out_shape=jax.ShapeDtypeStruct((2, N, HH), F32),
    )(e, binv_r)


def _final(oh, dinv_r, b2d):
    def body(o_ref, d_ref, b_ref, out_ref):
        dv = d_ref[...]
        bv = b_ref[...]
        out_ref[:, :HH] = o_ref[0] * dv + bv[:, :HH]
        out_ref[:, HH:] = o_ref[1] * dv + bv[:, HH:]

    return pl.pallas_call(
        body,
        grid=(_NI,),
        in_specs=[pl.BlockSpec((2, _RT, HH), lambda i: (0, i, 0)),
                  pl.BlockSpec((_RT, 1), lambda i: (i, 0)),
                  pl.BlockSpec((1, H), lambda i: (0, 0))],
        out_specs=pl.BlockSpec((_RT, H), lambda i: (i, 0)),
        out_shape=jax.ShapeDtypeStruct((N, H), F32),
    )(oh, dinv_r, b2d)


def kernel(hyperedge_index, hyperedge_weight, embedding, W0, b0, W1, b1, W2,
           b2):
    nidx = hyperedge_index[0]
    eidx = hyperedge_index[1]
    dinv, binv = _degrees(nidx, eidx, hyperedge_weight)
    dinv_r = dinv[:, None]
    binv_r = binv[:, None]

    y = _mm_first(embedding, W0, dinv)
    for Wn, bn in ((W1, b0), (W2, b1)):
        e = _seg_pass(y, nidx, eidx)
        z = _scale(e, binv_r)
        o = _seg_pass(z, eidx, nidx)
        y = _mm_mid(o, dinv_r, bn[None, :], Wn)
    e = _seg_pass(y, nidx, eidx)
    z = _scale(e, binv_r)
    o = _seg_pass(z, eidx, nidx)
    return _final(o, dinv_r, b2[None, :])


# speedup vs baseline: 1.0261x; 1.0076x over previous
import dataclasses
import functools

import jax
import jax.numpy as jnp
from jax import lax
from jax.experimental import pallas as pl
from jax.experimental.pallas import tpu as pltpu
from jax.experimental.pallas import tpu_sc as plsc

N = 10000
INC = 160000
H = 256
HH = 128
NSUB = 16
PER_TILE = INC // NSUB
CH = 80
NFULL = PER_TILE // CH
RQ = 4
SQ = 8
ROWS0 = 632
ROWSL = N - (NSUB - 1) * ROWS0
F32 = jnp.float32


def _mesh():
    return plsc.VectorSubcoreMesh(core_axis_name="c", subcore_axis_name="s",
                                  num_cores=2, num_subcores=NSUB)


def _seg_pass(src, gidx, sidx, token=None):
    extra = () if token is None else (token,)

    @functools.partial(
        pl.kernel,
        out_type=jax.ShapeDtypeStruct((2, N, HH), F32),
        mesh=_mesh(),
        scratch_types=[
            pltpu.VMEM((RQ, CH, HH), F32),
            pltpu.VMEM((SQ, CH), jnp.int32),
            pltpu.VMEM((SQ, CH), jnp.int32),
            pltpu.VMEM_SHARED((N, HH), F32),
            pltpu.SemaphoreType.DMA((RQ,)),
            pltpu.SemaphoreType.DMA((RQ,)),
            pltpu.SemaphoreType.DMA((SQ,)),
        ],
    )
    def k(gidx_hbm, sidx_hbm, src_hbm, *rest):
        (dst_hbm, rows, gbuf, sbuf, acc, sem_g, sem_s, sem_i) = rest[len(extra):]
        c = lax.axis_index("c")
        s = lax.axis_index("s")
        z16 = jnp.zeros((16,), F32)

        def phase_all(cc):
            base = s * PER_TILE

            def idx_cp(j, m):
                return (pltpu.make_async_copy(
                            gidx_hbm.at[pl.ds(base + j * CH, CH)],
                            gbuf.at[m], sem_i.at[m]),
                        pltpu.make_async_copy(
                            sidx_hbm.at[pl.ds(base + j * CH, CH)],
                            sbuf.at[m], sem_i.at[m]))

            def gat_cp(m8, m4):
                return pltpu.make_async_copy(src_hbm.at[cc].at[gbuf.at[m8]],
                                             rows.at[m4], sem_g.at[m4])

            def idx_start(j, m):
                a, b = idx_cp(j, m)
                a.start()
                b.start()

            def idx_wait(j, m):
                a, b = idx_cp(j, m)
                a.wait()
                b.wait()

            def sct_start(m8, m4):
                pltpu.async_copy(rows.at[m4], acc.at[sbuf.at[m8]],
                                 sem_s.at[m4], add=True)

            def sct_wait(m8, m4):
                pltpu.make_async_copy(rows.at[m4], acc.at[sbuf.at[m8]],
                                      sem_s.at[m4]).wait()

            def sched(kk, m8, head=False):
                m4 = m8 % 4
                gat_cp(m8, m4).wait()
                if not (head and isinstance(kk, int) and kk < 2):
                    sct_wait((m8 - 2) % SQ, (m4 - 2) % RQ)
                if not (isinstance(kk, int) and kk + 2 >= NFULL):
                    idx_wait(kk + 2, (m8 + 2) % SQ)
                    gat_cp((m8 + 2) % SQ, (m4 + 2) % RQ).start()
                if not (isinstance(kk, int) and kk + 4 >= NFULL):
                    idx_start(kk + 4, (m8 + 4) % SQ)
                sct_start(m8, m4)

            for j in range(4):
                idx_start(j, j)
            for j in range(2):
                idx_wait(j, j)
                gat_cp(j, j).start()

            @pl.loop(0, CH)
            def _(r):
                @pl.loop(0, HH, step=16)
                def _(j):
                    rows[RQ - 1, r, pl.ds(j, 16)] = z16

            def zero_stripe(roff, rlen):
                nf, rem = rlen // CH, rlen % CH
                for t in range(nf):
                    pltpu.sync_copy(rows.at[RQ - 1],
                                    acc.at[pl.ds(roff + t * CH, CH)])
                if rem:
                    pltpu.sync_copy(rows.at[RQ - 1].at[pl.ds(0, rem)],
                                    acc.at[pl.ds(roff + nf * CH, rem)])

            @pl.when(s < NSUB - 1)
            def _():
                zero_stripe(s * ROWS0, ROWS0)

            @pl.when(s == NSUB - 1)
            def _():
                zero_stripe((NSUB - 1) * ROWS0, ROWSL)

            plsc.subcore_barrier()

            for kk in range(8):
                sched(kk, kk, head=True)

            @pl.loop(8, 120, step=8)
            def _(k0):
                for d in range(8):
                    sched(k0 + d, d)

            for kk in range(120, NFULL):
                sched(kk, kk % SQ)

            sct_wait((NFULL - 2) % SQ, (NFULL - 2) % RQ)
            sct_wait((NFULL - 1) % SQ, (NFULL - 1) % RQ)

            plsc.subcore_barrier()

            @pl.when(s < NSUB - 1)
            def _():
                pltpu.sync_copy(acc.at[pl.ds(s * ROWS0, ROWS0)],
                                dst_hbm.at[cc].at[pl.ds(s * ROWS0, ROWS0)])

            @pl.when(s == NSUB - 1)
            def _():
                pltpu.sync_copy(
                    acc.at[pl.ds((NSUB - 1) * ROWS0, ROWSL)],
                    dst_hbm.at[cc].at[pl.ds((NSUB - 1) * ROWS0, ROWSL)])

        @pl.when(c == 0)
        def _():
            phase_all(0)

        @pl.when(c == 1)
        def _():
            phase_all(1)

    return k(gidx, sidx, src, *extra)


def _degrees(nidx, eidx, w):
    NPAD = 10240
    SPAN = NPAD // NSUB
    LASTD = N - (NSUB - 1) * SPAN

    cp = pltpu.CompilerParams()
    if "needs_layout_passes" in pltpu.CompilerParams.__dataclass_fields__:
        cp = dataclasses.replace(cp, needs_layout_passes=False)

    @functools.partial(
        pl.kernel,
        out_type=(jax.ShapeDtypeStruct((N,), F32),
                  jax.ShapeDtypeStruct((N,), F32)),
        mesh=_mesh(),
        compiler_params=cp,
        scratch_types=[
            pltpu.VMEM((PER_TILE,), jnp.int32),
            pltpu.VMEM((PER_TILE,), jnp.int32),
            pltpu.VMEM((N,), F32),
            pltpu.VMEM((NPAD,), F32),
            pltpu.VMEM_SHARED((NSUB, NPAD), F32),
            pltpu.VMEM((SPAN,), F32),
            pltpu.VMEM((SPAN,), F32),
        ],
    )
    def k(nidx_hbm, eidx_hbm, w_hbm, dinv_hbm, binv_hbm,
          ebuf, nbuf, wbuf, acc, stage, rbuf, abuf):
        c = lax.axis_index("c")
        s = lax.axis_index("s")
        z16 = jnp.zeros((16,), F32)
        base = s * PER_TILE
        pltpu.sync_copy(eidx_hbm.at[pl.ds(base, PER_TILE)], ebuf)

        @pl.loop(0, NPAD, step=16)
        def _(i):
            acc[pl.ds(i, 16)] = z16

        @pl.when(c == 0)
        def _():
            pltpu.sync_copy(nidx_hbm.at[pl.ds(base, PER_TILE)], nbuf)
            pltpu.sync_copy(w_hbm, wbuf)

            @pl.loop(0, PER_TILE, step=16)
            def _(i):
                e = ebuf[pl.ds(i, 16)]
                n = nbuf[pl.ds(i, 16)]
                wv = plsc.load_gather(wbuf, [e])
                plsc.addupdate_scatter(acc, [n], wv)

        @pl.when(c == 1)
        def _():
            o16 = jnp.full((16,), 1.0, F32)

            @pl.loop(0, PER_TILE, step=16)
            def _(i):
                e = ebuf[pl.ds(i, 16)]
                plsc.addupdate_scatter(acc, [e], o16)

        pltpu.sync_copy(acc, stage.at[s])
        plsc.subcore_barrier()

        def reduce_span(off, drain_len, out_hbm):
            @pl.loop(0, SPAN, step=16)
            def _(j):
                abuf[pl.ds(j, 16)] = z16

            @pl.loop(0, NSUB)
            def _(r):
                pltpu.sync_copy(stage.at[r].at[pl.ds(off, SPAN)], rbuf)

                @pl.loop(0, SPAN, step=16)
                def _(j):
                    abuf[pl.ds(j, 16)] = abuf[pl.ds(j, 16)] + rbuf[pl.ds(j, 16)]

            @pl.loop(0, SPAN, step=16)
            def _(j):
                v = abuf[pl.ds(j, 16)]
                zm = v == 0.0
                abuf[pl.ds(j, 16)] = jnp.where(zm, 0.0,
                                               1.0 / jnp.where(zm, 1.0, v))

            pltpu.sync_copy(abuf.at[pl.ds(0, drain_len)],
                            out_hbm.at[pl.ds(off, drain_len)])

        for cc, out_hbm in ((0, dinv_hbm), (1, binv_hbm)):
            @pl.when(jnp.logical_and(c == cc, s < NSUB - 1))
            def _():
                reduce_span(s * SPAN, SPAN, out_hbm)

            @pl.when(jnp.logical_and(c == cc, s == NSUB - 1))
            def _():
                reduce_span((NSUB - 1) * SPAN, LASTD, out_hbm)

    return k(nidx, eidx, w)


_RT = 2000
_NI = N // _RT


def _dot_t(x, wb):
    return lax.dot_general(x, wb, (((1,), (1,)), ((), ())),
                           precision=lax.Precision.DEFAULT,
                           preferred_element_type=F32)


def _mm_first(emb, W):
    def body(x_ref, w_ref, out_ref):
        x = x_ref[...]
        w = w_ref[...]
        for g in range(2):
            out_ref[g] = _dot_t(x, w[g * HH:(g + 1) * HH, :])

    return pl.pallas_call(
        body,
        grid=(_NI,),
        in_specs=[pl.BlockSpec((_RT, H), lambda i: (i, 0)),
                  pl.BlockSpec((H, H), lambda i: (0, 0))],
        out_specs=pl.BlockSpec((2, _RT, HH), lambda i: (0, i, 0)),
        out_shape=jax.ShapeDtypeStruct((2, N, HH), F32),
    )(emb, W)


def _mm_mid(oh, dinv_r, b2d, W):
    def body(o_ref, d_ref, b_ref, w_ref, out_ref):
        dv = d_ref[...]
        bv = b_ref[...]
        w = w_ref[...]
        x0 = jnp.maximum(o_ref[0] * dv + bv[:, :HH], 0.0)
        x1 = jnp.maximum(o_ref[1] * dv + bv[:, HH:], 0.0)
        x = jnp.concatenate([x0, x1], axis=1)
        for g in range(2):
            out_ref[g] = _dot_t(x, w[g * HH:(g + 1) * HH, :])

    return pl.pallas_call(
        body,
        grid=(_NI,),
        in_specs=[pl.BlockSpec((2, _RT, HH), lambda i: (0, i, 0)),
                  pl.BlockSpec((_RT, 1), lambda i: (i, 0)),
                  pl.BlockSpec((1, H), lambda i: (0, 0)),
                  pl.BlockSpec((H, H), lambda i: (0, 0))],
        out_specs=pl.BlockSpec((2, _RT, HH), lambda i: (0, i, 0)),
        out_shape=jax.ShapeDtypeStruct((2, N, HH), F32),
    )(oh, dinv_r, b2d, W)


def _scale(e, binv_r):
    def body(e_ref, s_ref, out_ref):
        out_ref[...] = e_ref[...] * s_ref[...][None, :, :]

    return pl.pallas_call(
        body,
        grid=(_NI,),
        in_specs=[pl.BlockSpec((2, _RT, HH), lambda i: (0, i, 0)),
                  pl.BlockSpec((_RT, 1), lambda i: (i, 0))],
        out_specs=pl.BlockSpec((2, _RT, HH), lambda i: (0, i, 0)),
        out_shape=jax.ShapeDtypeStruct((2, N, HH), F32),
    )(e, binv_r)


def _final(oh, dinv_r, b2d):
    def body(o_ref, d_ref, b_ref, out_ref):
        dv = d_ref[...]
        bv = b_ref[...]
        out_ref[:, :HH] = o_ref[0] * dv + bv[:, :HH]
        out_ref[:, HH:] = o_ref[1] * dv + bv[:, HH:]

    return pl.pallas_call(
        body,
        grid=(_NI,),
        in_specs=[pl.BlockSpec((2, _RT, HH), lambda i: (0, i, 0)),
                  pl.BlockSpec((_RT, 1), lambda i: (i, 0)),
                  pl.BlockSpec((1, H), lambda i: (0, 0))],
        out_specs=pl.BlockSpec((_RT, H), lambda i: (i, 0)),
        out_shape=jax.ShapeDtypeStruct((N, H), F32),
    )(oh, dinv_r, b2d)


def kernel(hyperedge_index, hyperedge_weight, embedding, W0, b0, W1, b1, W2,
           b2):
    nidx = hyperedge_index[0]
    eidx = hyperedge_index[1]
    dinv, binv = _degrees(nidx, eidx, hyperedge_weight)
    dinv_r = dinv[:, None]
    binv_r = binv[:, None]

    y = _mm_first(embedding, W0)
    first = True
    for Wn, bn in ((W1, b0), (W2, b1)):
        e = _seg_pass(y, nidx, eidx, token=dinv if first else None)
        first = False
        z = _scale(e, binv_r)
        o = _seg_pass(z, eidx, nidx)
        y = _mm_mid(o, dinv_r, bn[None, :], Wn)
    e = _seg_pass(y, nidx, eidx)
    z = _scale(e, binv_r)
    o = _seg_pass(z, eidx, nidx)
    return _final(o, dinv_r, b2[None, :])


# speedup vs baseline: 1.0282x; 1.0021x over previous
import dataclasses
import functools

import jax
import jax.numpy as jnp
from jax import lax
from jax.experimental import pallas as pl
from jax.experimental.pallas import tpu as pltpu
from jax.experimental.pallas import tpu_sc as plsc

N = 10000
INC = 160000
H = 256
HH = 128
NSUB = 16
PER_TILE = INC // NSUB
CH = 80
NFULL = PER_TILE // CH
RQ = 4
SQ = 8
ROWS0 = 632
ROWSL = N - (NSUB - 1) * ROWS0
F32 = jnp.float32


def _mesh():
    return plsc.VectorSubcoreMesh(core_axis_name="c", subcore_axis_name="s",
                                  num_cores=2, num_subcores=NSUB)


def _seg_pass(src, gidx, sidx, token=None):
    extra = () if token is None else (token,)

    @functools.partial(
        pl.kernel,
        out_type=jax.ShapeDtypeStruct((2, N, HH), F32),
        mesh=_mesh(),
        scratch_types=[
            pltpu.VMEM((RQ, CH, HH), F32),
            pltpu.VMEM((SQ, CH), jnp.int32),
            pltpu.VMEM((SQ, CH), jnp.int32),
            pltpu.VMEM_SHARED((N, HH), F32),
            pltpu.SemaphoreType.DMA((RQ,)),
            pltpu.SemaphoreType.DMA((RQ,)),
            pltpu.SemaphoreType.DMA((SQ,)),
        ],
    )
    def k(gidx_hbm, sidx_hbm, src_hbm, *rest):
        (dst_hbm, rows, gbuf, sbuf, acc, sem_g, sem_s, sem_i) = rest[len(extra):]
        c = lax.axis_index("c")
        s = lax.axis_index("s")
        z16 = jnp.zeros((16,), F32)

        def phase_all(cc):
            base = s * PER_TILE

            def idx_cp(j, m):
                return (pltpu.make_async_copy(
                            gidx_hbm.at[pl.ds(base + j * CH, CH)],
                            gbuf.at[m], sem_i.at[m]),
                        pltpu.make_async_copy(
                            sidx_hbm.at[pl.ds(base + j * CH, CH)],
                            sbuf.at[m], sem_i.at[m]))

            def gat_cp(m8, m4):
                return pltpu.make_async_copy(src_hbm.at[cc].at[gbuf.at[m8]],
                                             rows.at[m4], sem_g.at[m4])

            def idx_start(j, m):
                a, b = idx_cp(j, m)
                a.start()
                b.start()

            def idx_wait(j, m):
                a, b = idx_cp(j, m)
                a.wait()
                b.wait()

            def sct_start(m8, m4):
                pltpu.async_copy(rows.at[m4], acc.at[sbuf.at[m8]],
                                 sem_s.at[m4], add=True)

            def sct_wait(m8, m4):
                pltpu.make_async_copy(rows.at[m4], acc.at[sbuf.at[m8]],
                                      sem_s.at[m4]).wait()

            def sched(kk, m8, head=False):
                m4 = m8 % 4
                gat_cp(m8, m4).wait()
                if not (head and isinstance(kk, int) and kk < 2):
                    sct_wait((m8 - 2) % SQ, (m4 - 2) % RQ)
                if not (isinstance(kk, int) and kk + 2 >= NFULL):
                    idx_wait(kk + 2, (m8 + 2) % SQ)
                    gat_cp((m8 + 2) % SQ, (m4 + 2) % RQ).start()
                if not (isinstance(kk, int) and kk + 4 >= NFULL):
                    idx_start(kk + 4, (m8 + 4) % SQ)
                sct_start(m8, m4)

            for j in range(4):
                idx_start(j, j)
            for j in range(2):
                idx_wait(j, j)
                gat_cp(j, j).start()

            @pl.loop(0, CH)
            def _(r):
                @pl.loop(0, HH, step=16)
                def _(j):
                    rows[RQ - 1, r, pl.ds(j, 16)] = z16

            def zero_stripe(roff, rlen):
                nf, rem = rlen // CH, rlen % CH
                for t in range(nf):
                    pltpu.sync_copy(rows.at[RQ - 1],
                                    acc.at[pl.ds(roff + t * CH, CH)])
                if rem:
                    pltpu.sync_copy(rows.at[RQ - 1].at[pl.ds(0, rem)],
                                    acc.at[pl.ds(roff + nf * CH, rem)])

            @pl.when(s < NSUB - 1)
            def _():
                zero_stripe(s * ROWS0, ROWS0)

            @pl.when(s == NSUB - 1)
            def _():
                zero_stripe((NSUB - 1) * ROWS0, ROWSL)

            plsc.subcore_barrier()

            for kk in range(8):
                sched(kk, kk, head=True)

            @pl.loop(8, 120, step=8)
            def _(k0):
                for d in range(8):
                    sched(k0 + d, d)

            for kk in range(120, NFULL):
                sched(kk, kk % SQ)

            sct_wait((NFULL - 2) % SQ, (NFULL - 2) % RQ)
            sct_wait((NFULL - 1) % SQ, (NFULL - 1) % RQ)

            plsc.subcore_barrier()

            @pl.when(s < NSUB - 1)
            def _():
                pltpu.sync_copy(acc.at[pl.ds(s * ROWS0, ROWS0)],
                                dst_hbm.at[cc].at[pl.ds(s * ROWS0, ROWS0)])

            @pl.when(s == NSUB - 1)
            def _():
                pltpu.sync_copy(
                    acc.at[pl.ds((NSUB - 1) * ROWS0, ROWSL)],
                    dst_hbm.at[cc].at[pl.ds((NSUB - 1) * ROWS0, ROWSL)])

        @pl.when(c == 0)
        def _():
            phase_all(0)

        @pl.when(c == 1)
        def _():
            phase_all(1)

    return k(gidx, sidx, src, *extra)


def _degrees(nidx, eidx, w):
    NPAD = 10240
    SPAN = NPAD // NSUB
    LASTD = N - (NSUB - 1) * SPAN

    cp = pltpu.CompilerParams()
    if "needs_layout_passes" in pltpu.CompilerParams.__dataclass_fields__:
        cp = dataclasses.replace(cp, needs_layout_passes=False)

    @functools.partial(
        pl.kernel,
        out_type=(jax.ShapeDtypeStruct((N,), F32),
                  jax.ShapeDtypeStruct((N,), F32)),
        mesh=_mesh(),
        compiler_params=cp,
        scratch_types=[
            pltpu.VMEM((PER_TILE,), jnp.int32),
            pltpu.VMEM((PER_TILE,), jnp.int32),
            pltpu.VMEM((N,), F32),
            pltpu.VMEM((NPAD,), F32),
            pltpu.VMEM_SHARED((NSUB, NPAD), F32),
            pltpu.VMEM((2, SPAN), F32),
            pltpu.VMEM((SPAN,), F32),
            pltpu.SemaphoreType.DMA,
            pltpu.SemaphoreType.DMA,
            pltpu.SemaphoreType.DMA,
            pltpu.SemaphoreType.DMA((2,)),
        ],
    )
    def k(nidx_hbm, eidx_hbm, w_hbm, dinv_hbm, binv_hbm,
          ebuf, nbuf, wbuf, acc, stage, rbuf, abuf, sem_e, sem_n, sem_w,
          sem_r):
        c = lax.axis_index("c")
        s = lax.axis_index("s")
        z16 = jnp.zeros((16,), F32)
        base = s * PER_TILE
        cpe = pltpu.async_copy(eidx_hbm.at[pl.ds(base, PER_TILE)], ebuf,
                               sem_e)
        cpn = pltpu.async_copy(nidx_hbm.at[pl.ds(base, PER_TILE)], nbuf,
                               sem_n)
        cpw = pltpu.async_copy(w_hbm, wbuf, sem_w)

        @pl.loop(0, NPAD, step=16)
        def _(i):
            acc[pl.ds(i, 16)] = z16

        cpe.wait()
        cpn.wait()
        cpw.wait()

        @pl.when(c == 0)
        def _():
            @pl.loop(0, PER_TILE, step=16)
            def _(i):
                e = ebuf[pl.ds(i, 16)]
                n = nbuf[pl.ds(i, 16)]
                wv = plsc.load_gather(wbuf, [e])
                plsc.addupdate_scatter(acc, [n], wv)

        @pl.when(c == 1)
        def _():
            o16 = jnp.full((16,), 1.0, F32)

            @pl.loop(0, PER_TILE, step=16)
            def _(i):
                e = ebuf[pl.ds(i, 16)]
                plsc.addupdate_scatter(acc, [e], o16)

        pltpu.sync_copy(acc, stage.at[s])
        plsc.subcore_barrier()

        def reduce_span(off, drain_len, out_hbm):
            @pl.loop(0, SPAN, step=16)
            def _(j):
                abuf[pl.ds(j, 16)] = z16

            def row_cp(r, m):
                return pltpu.make_async_copy(
                    stage.at[r].at[pl.ds(off, SPAN)], rbuf.at[m], sem_r.at[m])

            row_cp(0, 0).start()
            for m0 in range(0, NSUB, 2):
                for m in range(2):
                    r = m0 + m
                    row_cp(r, m).wait()
                    if r + 1 < NSUB:
                        row_cp(r + 1, (m + 1) % 2).start()

                    @pl.loop(0, SPAN, step=16)
                    def _(j):
                        abuf[pl.ds(j, 16)] = (abuf[pl.ds(j, 16)]
                                              + rbuf[m, pl.ds(j, 16)])

            @pl.loop(0, SPAN, step=16)
            def _(j):
                v = abuf[pl.ds(j, 16)]
                zm = v == 0.0
                abuf[pl.ds(j, 16)] = jnp.where(zm, 0.0,
                                               1.0 / jnp.where(zm, 1.0, v))

            pltpu.sync_copy(abuf.at[pl.ds(0, drain_len)],
                            out_hbm.at[pl.ds(off, drain_len)])

        for cc, out_hbm in ((0, dinv_hbm), (1, binv_hbm)):
            @pl.when(jnp.logical_and(c == cc, s < NSUB - 1))
            def _():
                reduce_span(s * SPAN, SPAN, out_hbm)

            @pl.when(jnp.logical_and(c == cc, s == NSUB - 1))
            def _():
                reduce_span((NSUB - 1) * SPAN, LASTD, out_hbm)

    return k(nidx, eidx, w)


_RT = 2000
_NI = N // _RT


def _dot_t(x, wb):
    return lax.dot_general(x, wb, (((1,), (1,)), ((), ())),
                           precision=lax.Precision.DEFAULT,
                           preferred_element_type=F32)


def _mm_first(emb, W):
    def body(x_ref, w_ref, out_ref):
        x = x_ref[...]
        w = w_ref[...]
        for g in range(2):
            out_ref[g] = _dot_t(x, w[g * HH:(g + 1) * HH, :])

    return pl.pallas_call(
        body,
        grid=(_NI,),
        in_specs=[pl.BlockSpec((_RT, H), lambda i: (i, 0)),
                  pl.BlockSpec((H, H), lambda i: (0, 0))],
        out_specs=pl.BlockSpec((2, _RT, HH), lambda i: (0, i, 0)),
        out_shape=jax.ShapeDtypeStruct((2, N, HH), F32),
    )(emb, W)


def _mm_mid(oh, dinv_r, b2d, W):
    def body(o_ref, d_ref, b_ref, w_ref, out_ref):
        dv = d_ref[...]
        bv = b_ref[...]
        w = w_ref[...]
        x0 = jnp.maximum(o_ref[0] * dv + bv[:, :HH], 0.0)
        x1 = jnp.maximum(o_ref[1] * dv + bv[:, HH:], 0.0)
        x = jnp.concatenate([x0, x1], axis=1)
        for g in range(2):
            out_ref[g] = _dot_t(x, w[g * HH:(g + 1) * HH, :])

    return pl.pallas_call(
        body,
        grid=(_NI,),
        in_specs=[pl.BlockSpec((2, _RT, HH), lambda i: (0, i, 0)),
                  pl.BlockSpec((_RT, 1), lambda i: (i, 0)),
                  pl.BlockSpec((1, H), lambda i: (0, 0)),
                  pl.BlockSpec((H, H), lambda i: (0, 0))],
        out_specs=pl.BlockSpec((2, _RT, HH), lambda i: (0, i, 0)),
        out_shape=jax.ShapeDtypeStruct((2, N, HH), F32),
    )(oh, dinv_r, b2d, W)


def _scale(e, binv_r):
    def body(e_ref, s_ref, out_ref):
        out_ref[...] = e_ref[...] * s_ref[...][None, :, :]

    return pl.pallas_call(
        body,
        grid=(_NI,),
        in_specs=[pl.BlockSpec((2, _RT, HH), lambda i: (0, i, 0)),
                  pl.BlockSpec((_RT, 1), lambda i: (i, 0))],
        out_specs=pl.BlockSpec((2, _RT, HH), lambda i: (0, i, 0)),
        out_shape=jax.ShapeDtypeStruct((2, N, HH), F32),
    )(e, binv_r)


def _final(oh, dinv_r, b2d):
    def body(o_ref, d_ref, b_ref, out_ref):
        dv = d_ref[...]
        bv = b_ref[...]
        out_ref[:, :HH] = o_ref[0] * dv + bv[:, :HH]
        out_ref[:, HH:] = o_ref[1] * dv + bv[:, HH:]

    return pl.pallas_call(
        body,
        grid=(_NI,),
        in_specs=[pl.BlockSpec((2, _RT, HH), lambda i: (0, i, 0)),
                  pl.BlockSpec((_RT, 1), lambda i: (i, 0)),
                  pl.BlockSpec((1, H), lambda i: (0, 0))],
        out_specs=pl.BlockSpec((_RT, H), lambda i: (i, 0)),
        out_shape=jax.ShapeDtypeStruct((N, H), F32),
    )(oh, dinv_r, b2d)


def kernel(hyperedge_index, hyperedge_weight, embedding, W0, b0, W1, b1, W2,
           b2):
    nidx = hyperedge_index[0]
    eidx = hyperedge_index[1]
    dinv, binv = _degrees(nidx, eidx, hyperedge_weight)
    dinv_r = dinv[:, None]
    binv_r = binv[:, None]

    y = _mm_first(embedding, W0)
    first = True
    for Wn, bn in ((W1, b0), (W2, b1)):
        e = _seg_pass(y, nidx, eidx, token=dinv if first else None)
        first = False
        z = _scale(e, binv_r)
        o = _seg_pass(z, eidx, nidx)
        y = _mm_mid(o, dinv_r, bn[None, :], Wn)
    e = _seg_pass(y, nidx, eidx)
    z = _scale(e, binv_r)
    o = _seg_pass(z, eidx, nidx)
    return _final(o, dinv_r, b2[None, :])
